# baseline (speedup 1.0000x reference)
import functools

import jax
import jax.numpy as jnp
from jax import lax
from jax.experimental import pallas as pl
from jax.experimental.pallas import tpu as pltpu
from jax.experimental.pallas import tpu_sc as plsc

N = 10000
E = 320000
IN_FEATS = 128
HID = 32

NC = 2
NS = 16
NW = NC * NS
CH = 128
CPT = 80
EPT = CH * CPT
E_PAD = NW * EPT
N_PAD = 10240
ROWS_PER_TILE = N_PAD // NS
STAGE_ROWS = 160
N_SUB = ROWS_PER_TILE // STAGE_ROWS

_mesh = plsc.VectorSubcoreMesh(core_axis_name="c", subcore_axis_name="s")


@functools.partial(
    pl.kernel,
    mesh=_mesh,
    compiler_params=pltpu.CompilerParams(use_tc_tiling_on_sc=False),
    out_type=jax.ShapeDtypeStruct((NC, N_PAD, HID), jnp.float32),
    scratch_types=[
        pltpu.VMEM((CPT, CH), jnp.int32),
        pltpu.VMEM((CPT, CH), jnp.int32),
        pltpu.VMEM((CH, HID), jnp.float32),
        pltpu.VMEM((CH, HID), jnp.float32),
        pltpu.VMEM((STAGE_ROWS, HID), jnp.float32),
        pltpu.VMEM_SHARED((N_PAD, HID), jnp.float32),
        pltpu.SemaphoreType.DMA,
        pltpu.SemaphoreType.DMA,
        pltpu.SemaphoreType.DMA,
        pltpu.SemaphoreType.DMA,
    ],
)
def _sc_message_pass(h_hbm, src_hbm, dst_hbm, zeros_hbm, out_hbm,
                     srcv, dstv, rows_a, rows_b, stage_v, agg_sh,
                     sem_a, sem_b, ssem_a, ssem_b):
    cid = lax.axis_index("c")
    sid = lax.axis_index("s")
    wid = sid * NC + cid
    r0 = sid * ROWS_PER_TILE
    pltpu.sync_copy(zeros_hbm, stage_v)
    for k in range(N_SUB):
        pltpu.sync_copy(stage_v,
                        agg_sh.at[pl.ds(r0 + k * STAGE_ROWS, STAGE_ROWS)])
    pltpu.sync_copy(src_hbm.at[wid], srcv)
    pltpu.sync_copy(dst_hbm.at[wid], dstv)
    plsc.subcore_barrier()

    def g(j, buf, sem):
        return pltpu.make_async_copy(h_hbm.at[srcv.at[j]], buf, sem)

    def s(j, buf, sem):
        return pltpu.make_async_copy(buf, agg_sh.at[dstv.at[j]], sem)

    g(0, rows_a, sem_a).start()
    g(1, rows_b, sem_b).start()

    def pair(i, c):
        j = 2 * i
        g(j, rows_a, sem_a).wait()
        s(j, rows_a, ssem_a).start(add=True)
        g(j + 1, rows_b, sem_b).wait()
        s(j + 1, rows_b, ssem_b).start(add=True)
        s(j, rows_a, ssem_a).wait()
        g(j + 2, rows_a, sem_a).start()
        s(j + 1, rows_b, ssem_b).wait()
        g(j + 3, rows_b, sem_b).start()
        return c

    lax.fori_loop(0, CPT // 2 - 1, pair, 0)
    g(CPT - 2, rows_a, sem_a).wait()
    s(CPT - 2, rows_a, ssem_a).start(add=True)
    g(CPT - 1, rows_b, sem_b).wait()
    s(CPT - 1, rows_b, ssem_b).start(add=True)
    s(CPT - 2, rows_a, ssem_a).wait()
    s(CPT - 1, rows_b, ssem_b).wait()
    plsc.subcore_barrier()
    for k in range(N_SUB):
        sub = pl.ds(r0 + k * STAGE_ROWS, STAGE_ROWS)
        pltpu.sync_copy(agg_sh.at[sub], stage_v)
        pltpu.sync_copy(stage_v, out_hbm.at[cid, sub])


RB = 1000


def _tc1_body(x_ref, do_ref, di_ref, w_ref, h_ref, norm_ref):
    do_p = do_ref[...]
    di_p = di_ref[...]
    dout = (do_p[0] + do_p[1])[:, 0:1]
    din = (di_p[0] + di_p[1])[:, 0:1]
    nsrc = jnp.where(dout > 0.0, lax.rsqrt(jnp.maximum(dout, 1.0)), 0.0)
    ndst = jnp.where(din > 0.0, lax.rsqrt(jnp.maximum(din, 1.0)), 0.0)
    norm_ref[...] = jnp.concatenate([nsrc, ndst], axis=1)
    xs = x_ref[...] * nsrc
    h_ref[...] = jnp.dot(xs, w_ref[...], preferred_element_type=jnp.float32)


def _tc2_body(p_ref, norm_ref, b1_ref, w2_ref, h2_ref):
    p = p_ref[...]
    norm = norm_ref[...]
    h = (p[0] + p[1]) * norm[:, 1:2] + b1_ref[...]
    h = jnp.maximum(h, 0.0)
    h = h * norm[:, 0:1]
    h2_ref[...] = jnp.dot(h, w2_ref[...], preferred_element_type=jnp.float32)


def _tc3_body(p_ref, norm_ref, b2_ref, o_ref):
    p = p_ref[...]
    o_ref[...] = (p[0] + p[1]) * norm_ref[...][:, 1:2] + b2_ref[...]


_G = N // RB

_tc1 = pl.pallas_call(
    _tc1_body,
    grid=(_G,),
    in_specs=[
        pl.BlockSpec((RB, IN_FEATS), lambda i: (i, 0)),
        pl.BlockSpec((NC, RB, HID), lambda i: (0, i, 0)),
        pl.BlockSpec((NC, RB, HID), lambda i: (0, i, 0)),
        pl.BlockSpec((IN_FEATS, HID), lambda i: (0, 0)),
    ],
    out_specs=[
        pl.BlockSpec((RB, HID), lambda i: (i, 0)),
        pl.BlockSpec((RB, 2), lambda i: (i, 0)),
    ],
    out_shape=[
        jax.ShapeDtypeStruct((N_PAD, HID), jnp.float32),
        jax.ShapeDtypeStruct((N, 2), jnp.float32),
    ],
)

_tc2 = pl.pallas_call(
    _tc2_body,
    grid=(_G,),
    in_specs=[
        pl.BlockSpec((NC, RB, HID), lambda i: (0, i, 0)),
        pl.BlockSpec((RB, 2), lambda i: (i, 0)),
        pl.BlockSpec((1, HID), lambda i: (0, 0)),
        pl.BlockSpec((HID, HID), lambda i: (0, 0)),
    ],
    out_specs=pl.BlockSpec((RB, HID), lambda i: (i, 0)),
    out_shape=jax.ShapeDtypeStruct((N_PAD, HID), jnp.float32),
)

_tc3 = pl.pallas_call(
    _tc3_body,
    grid=(_G,),
    in_specs=[
        pl.BlockSpec((NC, RB, HID), lambda i: (0, i, 0)),
        pl.BlockSpec((RB, 2), lambda i: (i, 0)),
        pl.BlockSpec((1, HID), lambda i: (0, 0)),
    ],
    out_specs=pl.BlockSpec((RB, HID), lambda i: (i, 0)),
    out_shape=jax.ShapeDtypeStruct((N, HID), jnp.float32),
)


def kernel(x, edge_index, W1, b1, W2, b2):
    src = edge_index[0]
    dst = edge_index[1]
    pad_idx = (jnp.arange(E_PAD - E, dtype=jnp.int32) % (N_PAD - N)) + N
    src_p = jnp.concatenate([src, pad_idx]).reshape(NW, CPT, CH)
    dst_p = jnp.concatenate([dst, pad_idx]).reshape(NW, CPT, CH)

    zeros_f = jnp.zeros((STAGE_ROWS, HID), jnp.float32)
    ones_t = jnp.ones((N_PAD, HID), jnp.float32)

    din_f = _sc_message_pass(ones_t, src_p, dst_p, zeros_f)
    dout_f = _sc_message_pass(ones_t, dst_p, src_p, zeros_f)

    h1, norms = _tc1(x, dout_f, din_f, W1)
    agg1 = _sc_message_pass(h1, src_p, dst_p, zeros_f)
    h2 = _tc2(agg1, norms, b1.reshape(1, HID), W2)
    agg2 = _sc_message_pass(h2, src_p, dst_p, zeros_f)
    out = _tc3(agg2, norms, b2.reshape(1, HID))
    return out


# speedup vs baseline: 1.2708x; 1.2708x over previous
import functools

import jax
import jax.numpy as jnp
from jax import lax
from jax.experimental import pallas as pl
from jax.experimental.pallas import tpu as pltpu
from jax.experimental.pallas import tpu_sc as plsc

N = 10000
E = 320000
IN_FEATS = 128
HID = 32

NC = 2
NS = 16
NW = NC * NS
CH = 128
CPT = 79
EPT = CH * CPT
E_PAD = NW * EPT
N_PAD = 10240
ROWS_PER_TILE = N_PAD // NS
STAGE_ROWS = 160
N_SUB = ROWS_PER_TILE // STAGE_ROWS

_mesh = plsc.VectorSubcoreMesh(core_axis_name="c", subcore_axis_name="s")


@functools.partial(
    pl.kernel,
    mesh=_mesh,
    compiler_params=pltpu.CompilerParams(use_tc_tiling_on_sc=False, needs_layout_passes=False),
    out_type=jax.ShapeDtypeStruct((NC, N_PAD, HID), jnp.float32),
    scratch_types=[
        pltpu.VMEM((CPT, CH), jnp.int32),
        pltpu.VMEM((CPT, CH), jnp.int32),
        pltpu.VMEM((CH, HID), jnp.float32),
        pltpu.VMEM((CH, HID), jnp.float32),
        pltpu.VMEM((STAGE_ROWS, HID), jnp.float32),
        pltpu.VMEM((16,), jnp.float32),
        pltpu.VMEM_SHARED((N_PAD, HID), jnp.float32),
        pltpu.SemaphoreType.DMA,
        pltpu.SemaphoreType.DMA,
    ],
)
def _sc_message_pass(h_hbm, src_hbm, dst_hbm, zeros_hbm, ones_hbm, mode_hbm,
                     out_hbm, srcv, dstv, rows_a, rows_b, stage_v, mode_v,
                     agg_sh, sem_a, sem_b):
    cid = lax.axis_index("c")
    sid = lax.axis_index("s")
    wid = sid * NC + cid
    r0 = sid * ROWS_PER_TILE
    pltpu.sync_copy(zeros_hbm, stage_v)
    for k in range(N_SUB):
        pltpu.sync_copy(stage_v,
                        agg_sh.at[pl.ds(r0 + k * STAGE_ROWS, STAGE_ROWS)])
    pltpu.sync_copy(mode_hbm, mode_v)
    pltpu.sync_copy(dst_hbm.at[wid], dstv)
    is_deg = jnp.sum(mode_v[...]) > 0.5
    plsc.subcore_barrier()

    @pl.when(jnp.logical_not(is_deg))
    def _mp_loop():
        pltpu.sync_copy(src_hbm.at[wid], srcv)
        pltpu.make_async_copy(h_hbm.at[srcv.at[0]], rows_a, sem_a).start()

        def pair(i, c):
            j = 1 + 2 * i
            pltpu.make_async_copy(h_hbm.at[srcv.at[j]], rows_b,
                                  sem_b).start()
            pltpu.make_async_copy(h_hbm.at[srcv.at[j - 1]], rows_a,
                                  sem_a).wait()
            pltpu.sync_copy(rows_a, agg_sh.at[dstv.at[j - 1]], add=True)
            pltpu.make_async_copy(h_hbm.at[srcv.at[j + 1]], rows_a,
                                  sem_a).start()
            pltpu.make_async_copy(h_hbm.at[srcv.at[j]], rows_b,
                                  sem_b).wait()
            pltpu.sync_copy(rows_b, agg_sh.at[dstv.at[j]], add=True)
            return c

        lax.fori_loop(0, (CPT - 1) // 2, pair, 0)
        pltpu.make_async_copy(h_hbm.at[srcv.at[CPT - 1]], rows_a,
                              sem_a).wait()
        pltpu.sync_copy(rows_a, agg_sh.at[dstv.at[CPT - 1]], add=True)

    @pl.when(is_deg)
    def _deg_loop():
        pltpu.sync_copy(ones_hbm, rows_a)

        def dpair(i, c):
            j = 2 * i
            sa = pltpu.make_async_copy(rows_a, agg_sh.at[dstv.at[j]], sem_a)
            sb = pltpu.make_async_copy(rows_a, agg_sh.at[dstv.at[j + 1]],
                                       sem_b)
            sa.start(add=True)
            sb.start(add=True)
            sa.wait()
            sb.wait()
            return c

        lax.fori_loop(0, (CPT - 1) // 2, dpair, 0)
        pltpu.sync_copy(rows_a, agg_sh.at[dstv.at[CPT - 1]], add=True)

    plsc.subcore_barrier()
    for k in range(N_SUB):
        sub = pl.ds(r0 + k * STAGE_ROWS, STAGE_ROWS)
        pltpu.sync_copy(agg_sh.at[sub], stage_v)
        pltpu.sync_copy(stage_v, out_hbm.at[cid, sub])


RB = 1000


def _tc1_body(x_ref, do_ref, di_ref, w_ref, h_ref, norm_ref):
    do_p = do_ref[...]
    di_p = di_ref[...]
    dout = (do_p[0] + do_p[1])[:, 0:1]
    din = (di_p[0] + di_p[1])[:, 0:1]
    nsrc = jnp.where(dout > 0.0, lax.rsqrt(jnp.maximum(dout, 1.0)), 0.0)
    ndst = jnp.where(din > 0.0, lax.rsqrt(jnp.maximum(din, 1.0)), 0.0)
    norm_ref[...] = jnp.concatenate([nsrc, ndst], axis=1)
    xs = x_ref[...] * nsrc
    h_ref[...] = jnp.dot(xs, w_ref[...], preferred_element_type=jnp.float32)


def _tc2_body(p_ref, norm_ref, b1_ref, w2_ref, h2_ref):
    p = p_ref[...]
    norm = norm_ref[...]
    h = (p[0] + p[1]) * norm[:, 1:2] + b1_ref[...]
    h = jnp.maximum(h, 0.0)
    h = h * norm[:, 0:1]
    h2_ref[...] = jnp.dot(h, w2_ref[...], preferred_element_type=jnp.float32)


def _tc3_body(p_ref, norm_ref, b2_ref, o_ref):
    p = p_ref[...]
    o_ref[...] = (p[0] + p[1]) * norm_ref[...][:, 1:2] + b2_ref[...]


_G = N // RB

_tc1 = pl.pallas_call(
    _tc1_body,
    grid=(_G,),
    in_specs=[
        pl.BlockSpec((RB, IN_FEATS), lambda i: (i, 0)),
        pl.BlockSpec((NC, RB, HID), lambda i: (0, i, 0)),
        pl.BlockSpec((NC, RB, HID), lambda i: (0, i, 0)),
        pl.BlockSpec((IN_FEATS, HID), lambda i: (0, 0)),
    ],
    out_specs=[
        pl.BlockSpec((RB, HID), lambda i: (i, 0)),
        pl.BlockSpec((RB, 2), lambda i: (i, 0)),
    ],
    out_shape=[
        jax.ShapeDtypeStruct((N_PAD, HID), jnp.float32),
        jax.ShapeDtypeStruct((N, 2), jnp.float32),
    ],
)

_tc2 = pl.pallas_call(
    _tc2_body,
    grid=(_G,),
    in_specs=[
        pl.BlockSpec((NC, RB, HID), lambda i: (0, i, 0)),
        pl.BlockSpec((RB, 2), lambda i: (i, 0)),
        pl.BlockSpec((1, HID), lambda i: (0, 0)),
        pl.BlockSpec((HID, HID), lambda i: (0, 0)),
    ],
    out_specs=pl.BlockSpec((RB, HID), lambda i: (i, 0)),
    out_shape=jax.ShapeDtypeStruct((N_PAD, HID), jnp.float32),
)

_tc3 = pl.pallas_call(
    _tc3_body,
    grid=(_G,),
    in_specs=[
        pl.BlockSpec((NC, RB, HID), lambda i: (0, i, 0)),
        pl.BlockSpec((RB, 2), lambda i: (i, 0)),
        pl.BlockSpec((1, HID), lambda i: (0, 0)),
    ],
    out_specs=pl.BlockSpec((RB, HID), lambda i: (i, 0)),
    out_shape=jax.ShapeDtypeStruct((N, HID), jnp.float32),
)


def kernel(x, edge_index, W1, b1, W2, b2):
    src = edge_index[0]
    dst = edge_index[1]
    pad_idx = (jnp.arange(E_PAD - E, dtype=jnp.int32) % (N_PAD - N)) + N
    src_p = jnp.concatenate([src, pad_idx]).reshape(NW, CPT, CH)
    dst_p = jnp.concatenate([dst, pad_idx]).reshape(NW, CPT, CH)

    zeros_f = jnp.zeros((STAGE_ROWS, HID), jnp.float32)
    ones_r = jnp.ones((CH, HID), jnp.float32)
    dummy_t = jnp.zeros((N_PAD, HID), jnp.float32)
    mode_mp = jnp.zeros((16,), jnp.float32)
    mode_deg = jnp.ones((16,), jnp.float32)

    din_f = _sc_message_pass(dummy_t, src_p, dst_p, zeros_f, ones_r,
                             mode_deg)
    dout_f = _sc_message_pass(dummy_t, dst_p, src_p, zeros_f, ones_r,
                              mode_deg)

    h1, norms = _tc1(x, dout_f, din_f, W1)
    agg1 = _sc_message_pass(h1, src_p, dst_p, zeros_f, ones_r, mode_mp)
    h2 = _tc2(agg1, norms, b1.reshape(1, HID), W2)
    agg2 = _sc_message_pass(h2, src_p, dst_p, zeros_f, ones_r, mode_mp)
    out = _tc3(agg2, norms, b2.reshape(1, HID))
    return out


# speedup vs baseline: 1.3484x; 1.0611x over previous
import functools

import jax
import jax.numpy as jnp
from jax import lax
from jax.experimental import pallas as pl
from jax.experimental.pallas import tpu as pltpu
from jax.experimental.pallas import tpu_sc as plsc

N = 10000
E = 320000
IN_FEATS = 128
HID = 32

NC = 2
NS = 16
NW = NC * NS
CH = 128
CPT = 79
EPT = CH * CPT
E_PAD = NW * EPT
N_PAD = 10240
ROWS_PER_TILE = N_PAD // NS
STAGE_ROWS = 160
N_SUB = ROWS_PER_TILE // STAGE_ROWS

_mesh = plsc.VectorSubcoreMesh(core_axis_name="c", subcore_axis_name="s")


@functools.partial(
    pl.kernel,
    mesh=_mesh,
    compiler_params=pltpu.CompilerParams(use_tc_tiling_on_sc=False, needs_layout_passes=False),
    out_type=jax.ShapeDtypeStruct((NC, N_PAD, HID), jnp.float32),
    scratch_types=[
        pltpu.VMEM((CPT, CH), jnp.int32),
        pltpu.VMEM((CPT, CH), jnp.int32),
        pltpu.VMEM((CH, HID), jnp.float32),
        pltpu.VMEM((CH, HID), jnp.float32),
        pltpu.VMEM((STAGE_ROWS, HID), jnp.float32),
        pltpu.VMEM((16,), jnp.float32),
        pltpu.VMEM_SHARED((N_PAD, HID), jnp.float32),
        pltpu.SemaphoreType.DMA,
        pltpu.SemaphoreType.DMA,
    ],
)
def _sc_message_pass(h_hbm, src_hbm, dst_hbm, zeros_hbm, ones_hbm, mode_hbm,
                     out_hbm, srcv, dstv, rows_a, rows_b, stage_v, mode_v,
                     agg_sh, sem_a, sem_b):
    cid = lax.axis_index("c")
    sid = lax.axis_index("s")
    wid = sid * NC + cid
    r0 = sid * ROWS_PER_TILE
    pltpu.sync_copy(zeros_hbm, stage_v)
    for k in range(N_SUB):
        pltpu.sync_copy(stage_v,
                        agg_sh.at[pl.ds(r0 + k * STAGE_ROWS, STAGE_ROWS)])
    pltpu.sync_copy(mode_hbm, mode_v)
    pltpu.sync_copy(dst_hbm.at[wid], dstv)
    is_deg = jnp.sum(mode_v[...]) > 0.5
    plsc.subcore_barrier()

    @pl.when(jnp.logical_not(is_deg))
    def _mp_loop():
        pltpu.sync_copy(src_hbm.at[wid], srcv)
        pltpu.make_async_copy(h_hbm.at[srcv.at[0]], rows_a, sem_a).start()

        def pair(i, c):
            j = 1 + 2 * i
            pltpu.make_async_copy(h_hbm.at[srcv.at[j]], rows_b,
                                  sem_b).start()
            pltpu.make_async_copy(h_hbm.at[srcv.at[j - 1]], rows_a,
                                  sem_a).wait()
            pltpu.sync_copy(rows_a, agg_sh.at[dstv.at[j - 1]], add=True)
            pltpu.make_async_copy(h_hbm.at[srcv.at[j + 1]], rows_a,
                                  sem_a).start()
            pltpu.make_async_copy(h_hbm.at[srcv.at[j]], rows_b,
                                  sem_b).wait()
            pltpu.sync_copy(rows_b, agg_sh.at[dstv.at[j]], add=True)
            return c

        lax.fori_loop(0, (CPT - 1) // 2, pair, 0)
        pltpu.make_async_copy(h_hbm.at[srcv.at[CPT - 1]], rows_a,
                              sem_a).wait()
        pltpu.sync_copy(rows_a, agg_sh.at[dstv.at[CPT - 1]], add=True)

    @pl.when(is_deg)
    def _deg_loop():
        pltpu.sync_copy(ones_hbm, rows_a)

        def scatter_shard(idx_hbm, w):
            pltpu.sync_copy(idx_hbm.at[w], dstv)

            def dpair(i, c):
                j = 2 * i
                sa = pltpu.make_async_copy(rows_a, agg_sh.at[dstv.at[j]],
                                           sem_a)
                sb = pltpu.make_async_copy(rows_a, agg_sh.at[dstv.at[j + 1]],
                                           sem_b)
                sa.start(add=True)
                sb.start(add=True)
                sa.wait()
                sb.wait()
                return c

            lax.fori_loop(0, (CPT - 1) // 2, dpair, 0)
            pltpu.sync_copy(rows_a, agg_sh.at[dstv.at[CPT - 1]], add=True)

        @pl.when(cid == 0)
        def _in_degrees():
            scatter_shard(dst_hbm, sid)
            scatter_shard(dst_hbm, sid + NS)

        @pl.when(cid == 1)
        def _out_degrees():
            scatter_shard(src_hbm, sid)
            scatter_shard(src_hbm, sid + NS)

    plsc.subcore_barrier()
    for k in range(N_SUB):
        sub = pl.ds(r0 + k * STAGE_ROWS, STAGE_ROWS)
        pltpu.sync_copy(agg_sh.at[sub], stage_v)
        pltpu.sync_copy(stage_v, out_hbm.at[cid, sub])


RB = 1000


def _tc1_body(x_ref, deg_ref, w_ref, h_ref, norm_ref):
    d = deg_ref[...]
    din = d[0][:, 0:1]
    dout = d[1][:, 0:1]
    nsrc = jnp.where(dout > 0.0, lax.rsqrt(jnp.maximum(dout, 1.0)), 0.0)
    ndst = jnp.where(din > 0.0, lax.rsqrt(jnp.maximum(din, 1.0)), 0.0)
    norm_ref[...] = jnp.concatenate([nsrc, ndst], axis=1)
    xs = x_ref[...] * nsrc
    h_ref[...] = jnp.dot(xs, w_ref[...], preferred_element_type=jnp.float32)


def _tc2_body(p_ref, norm_ref, b1_ref, w2_ref, h2_ref):
    p = p_ref[...]
    norm = norm_ref[...]
    h = (p[0] + p[1]) * norm[:, 1:2] + b1_ref[...]
    h = jnp.maximum(h, 0.0)
    h = h * norm[:, 0:1]
    h2_ref[...] = jnp.dot(h, w2_ref[...], preferred_element_type=jnp.float32)


def _tc3_body(p_ref, norm_ref, b2_ref, o_ref):
    p = p_ref[...]
    o_ref[...] = (p[0] + p[1]) * norm_ref[...][:, 1:2] + b2_ref[...]


_G = N // RB

_tc1 = pl.pallas_call(
    _tc1_body,
    grid=(_G,),
    in_specs=[
        pl.BlockSpec((RB, IN_FEATS), lambda i: (i, 0)),
        pl.BlockSpec((NC, RB, HID), lambda i: (0, i, 0)),
        pl.BlockSpec((IN_FEATS, HID), lambda i: (0, 0)),
    ],
    out_specs=[
        pl.BlockSpec((RB, HID), lambda i: (i, 0)),
        pl.BlockSpec((RB, 2), lambda i: (i, 0)),
    ],
    out_shape=[
        jax.ShapeDtypeStruct((N_PAD, HID), jnp.float32),
        jax.ShapeDtypeStruct((N, 2), jnp.float32),
    ],
)

_tc2 = pl.pallas_call(
    _tc2_body,
    grid=(_G,),
    in_specs=[
        pl.BlockSpec((NC, RB, HID), lambda i: (0, i, 0)),
        pl.BlockSpec((RB, 2), lambda i: (i, 0)),
        pl.BlockSpec((1, HID), lambda i: (0, 0)),
        pl.BlockSpec((HID, HID), lambda i: (0, 0)),
    ],
    out_specs=pl.BlockSpec((RB, HID), lambda i: (i, 0)),
    out_shape=jax.ShapeDtypeStruct((N_PAD, HID), jnp.float32),
)

_tc3 = pl.pallas_call(
    _tc3_body,
    grid=(_G,),
    in_specs=[
        pl.BlockSpec((NC, RB, HID), lambda i: (0, i, 0)),
        pl.BlockSpec((RB, 2), lambda i: (i, 0)),
        pl.BlockSpec((1, HID), lambda i: (0, 0)),
    ],
    out_specs=pl.BlockSpec((RB, HID), lambda i: (i, 0)),
    out_shape=jax.ShapeDtypeStruct((N, HID), jnp.float32),
)


def kernel(x, edge_index, W1, b1, W2, b2):
    src = edge_index[0]
    dst = edge_index[1]
    pad_idx = (jnp.arange(E_PAD - E, dtype=jnp.int32) % (N_PAD - N)) + N
    src_p = jnp.concatenate([src, pad_idx]).reshape(NW, CPT, CH)
    dst_p = jnp.concatenate([dst, pad_idx]).reshape(NW, CPT, CH)

    zeros_f = jnp.zeros((STAGE_ROWS, HID), jnp.float32)
    ones_r = jnp.ones((CH, HID), jnp.float32)
    dummy_t = jnp.zeros((N_PAD, HID), jnp.float32)
    mode_mp = jnp.zeros((16,), jnp.float32)
    mode_deg = jnp.ones((16,), jnp.float32)

    degf = _sc_message_pass(dummy_t, src_p, dst_p, zeros_f, ones_r,
                            mode_deg)

    h1, norms = _tc1(x, degf, W1)
    agg1 = _sc_message_pass(h1, src_p, dst_p, zeros_f, ones_r, mode_mp)
    h2 = _tc2(agg1, norms, b1.reshape(1, HID), W2)
    agg2 = _sc_message_pass(h2, src_p, dst_p, zeros_f, ones_r, mode_mp)
    out = _tc3(agg2, norms, b2.reshape(1, HID))
    return out


# speedup vs baseline: 1.5453x; 1.1460x over previous
import functools

import jax
import jax.numpy as jnp
from jax import lax
from jax.experimental import pallas as pl
from jax.experimental.pallas import tpu as pltpu
from jax.experimental.pallas import tpu_sc as plsc

N = 10000
E = 320000
IN_FEATS = 128
HID = 32

NC = 2
NS = 16
NW = NC * NS
CH = 128
CPT = 79
EPT = CH * CPT
E_PAD = NW * EPT
N_PAD = 10240
ROWS_PER_TILE = N_PAD // NS
STAGE_ROWS = 160
N_SUB = ROWS_PER_TILE // STAGE_ROWS

_mesh = plsc.VectorSubcoreMesh(core_axis_name="c", subcore_axis_name="s")


@functools.partial(
    pl.kernel,
    mesh=_mesh,
    compiler_params=pltpu.CompilerParams(use_tc_tiling_on_sc=False, needs_layout_passes=False),
    out_type=jax.ShapeDtypeStruct((NC, N_PAD, HID), jnp.float32),
    scratch_types=[
        pltpu.VMEM((CPT, CH), jnp.int32),
        pltpu.VMEM((CPT, CH), jnp.int32),
        pltpu.VMEM((CH, HID), jnp.float32),
        pltpu.VMEM((CH, HID), jnp.float32),
        pltpu.VMEM((STAGE_ROWS, HID), jnp.float32),
        pltpu.VMEM((16,), jnp.float32),
        pltpu.VMEM_SHARED((N_PAD, HID), jnp.float32),
        pltpu.SemaphoreType.DMA,
        pltpu.SemaphoreType.DMA,
    ],
)
def _sc_message_pass(h_hbm, src_hbm, dst_hbm, zeros_hbm, ones_hbm, mode_hbm,
                     out_hbm, srcv, dstv, rows_a, rows_b, stage_v, mode_v,
                     agg_sh, sem_a, sem_b):
    cid = lax.axis_index("c")
    sid = lax.axis_index("s")
    wid = sid * NC + cid
    r0 = sid * ROWS_PER_TILE
    pltpu.sync_copy(zeros_hbm, stage_v)
    for k in range(N_SUB):
        pltpu.sync_copy(stage_v,
                        agg_sh.at[pl.ds(r0 + k * STAGE_ROWS, STAGE_ROWS)])
    pltpu.sync_copy(mode_hbm, mode_v)
    pltpu.sync_copy(dst_hbm.at[wid], dstv)
    is_deg = jnp.sum(mode_v[...]) > 0.5
    plsc.subcore_barrier()

    @pl.when(jnp.logical_not(is_deg))
    def _mp_loop():
        pltpu.sync_copy(src_hbm.at[wid], srcv)
        pltpu.make_async_copy(h_hbm.at[srcv.at[0]], rows_a, sem_a).start()

        def pair(i, c):
            j = 1 + 2 * i
            pltpu.make_async_copy(h_hbm.at[srcv.at[j]], rows_b,
                                  sem_b).start()
            pltpu.make_async_copy(h_hbm.at[srcv.at[j - 1]], rows_a,
                                  sem_a).wait()
            pltpu.sync_copy(rows_a, agg_sh.at[dstv.at[j - 1]], add=True)
            pltpu.make_async_copy(h_hbm.at[srcv.at[j + 1]], rows_a,
                                  sem_a).start()
            pltpu.make_async_copy(h_hbm.at[srcv.at[j]], rows_b,
                                  sem_b).wait()
            pltpu.sync_copy(rows_b, agg_sh.at[dstv.at[j]], add=True)
            return c

        lax.fori_loop(0, (CPT - 1) // 2, pair, 0)
        pltpu.make_async_copy(h_hbm.at[srcv.at[CPT - 1]], rows_a,
                              sem_a).wait()
        pltpu.sync_copy(rows_a, agg_sh.at[dstv.at[CPT - 1]], add=True)

    @pl.when(is_deg)
    def _deg_loop():
        pltpu.sync_copy(ones_hbm, rows_a)

        def scatter_shard(idx_hbm, w):
            pltpu.sync_copy(idx_hbm.at[w], dstv)

            def dpair(i, c):
                j = 2 * i
                sa = pltpu.make_async_copy(rows_a, agg_sh.at[dstv.at[j]],
                                           sem_a)
                sb = pltpu.make_async_copy(rows_a, agg_sh.at[dstv.at[j + 1]],
                                           sem_b)
                sa.start(add=True)
                sb.start(add=True)
                sa.wait()
                sb.wait()
                return c

            lax.fori_loop(0, (CPT - 1) // 2, dpair, 0)
            pltpu.sync_copy(rows_a, agg_sh.at[dstv.at[CPT - 1]], add=True)

        @pl.when(cid == 0)
        def _in_degrees():
            scatter_shard(dst_hbm, sid)
            scatter_shard(dst_hbm, sid + NS)

        @pl.when(cid == 1)
        def _out_degrees():
            scatter_shard(src_hbm, sid)
            scatter_shard(src_hbm, sid + NS)

    plsc.subcore_barrier()
    for k in range(N_SUB):
        sub = pl.ds(r0 + k * STAGE_ROWS, STAGE_ROWS)
        pltpu.sync_copy(agg_sh.at[sub], stage_v)
        pltpu.sync_copy(stage_v, out_hbm.at[cid, sub])


RB = 1000


def _gated_rsqrt(d):
    return jnp.where(d > 0.0, lax.rsqrt(jnp.maximum(d, 1.0)), 0.0)


def _tc1_body(x_ref, deg_ref, w_ref, h_ref):
    d = deg_ref[...]
    nsrc = _gated_rsqrt(d[1][:, 0:1])
    xs = x_ref[...] * nsrc
    h_ref[...] = jnp.dot(xs, w_ref[...], preferred_element_type=jnp.float32)


def _tc2_body(p_ref, din_ref, dout_ref, b1_ref, w2_ref, h2_ref):
    p = p_ref[...]
    ndst = _gated_rsqrt(din_ref[...][0])
    nsrc = _gated_rsqrt(dout_ref[...][0])
    h = (p[0] + p[1]) * ndst + b1_ref[...]
    h = jnp.maximum(h, 0.0)
    h = h * nsrc
    h2_ref[...] = jnp.dot(h, w2_ref[...], preferred_element_type=jnp.float32)


def _tc3_body(p_ref, din_ref, b2_ref, o_ref):
    p = p_ref[...]
    ndst = _gated_rsqrt(din_ref[...][0])
    o_ref[...] = (p[0] + p[1]) * ndst + b2_ref[...]


_G = N // RB
NF = N_PAD // 4
HF = 4 * HID
RBF = 512
_GF = NF // RBF

_tc1 = pl.pallas_call(
    _tc1_body,
    grid=(_G,),
    in_specs=[
        pl.BlockSpec((RB, IN_FEATS), lambda i: (i, 0)),
        pl.BlockSpec((NC, RB, HID), lambda i: (0, i, 0)),
        pl.BlockSpec((IN_FEATS, HID), lambda i: (0, 0)),
    ],
    out_specs=pl.BlockSpec((RB, HID), lambda i: (i, 0)),
    out_shape=jax.ShapeDtypeStruct((N_PAD, HID), jnp.float32),
)

_tc2 = pl.pallas_call(
    _tc2_body,
    grid=(_GF,),
    in_specs=[
        pl.BlockSpec((NC, RBF, HF), lambda i: (0, i, 0)),
        pl.BlockSpec((1, RBF, HF), lambda i: (0, i, 0)),
        pl.BlockSpec((1, RBF, HF), lambda i: (1, i, 0)),
        pl.BlockSpec((1, HF), lambda i: (0, 0)),
        pl.BlockSpec((HF, HF), lambda i: (0, 0)),
    ],
    out_specs=pl.BlockSpec((RBF, HF), lambda i: (i, 0)),
    out_shape=jax.ShapeDtypeStruct((NF, HF), jnp.float32),
)

_tc3 = pl.pallas_call(
    _tc3_body,
    grid=(_GF,),
    in_specs=[
        pl.BlockSpec((NC, RBF, HF), lambda i: (0, i, 0)),
        pl.BlockSpec((1, RBF, HF), lambda i: (0, i, 0)),
        pl.BlockSpec((1, HF), lambda i: (0, 0)),
    ],
    out_specs=pl.BlockSpec((RBF, HF), lambda i: (i, 0)),
    out_shape=jax.ShapeDtypeStruct((NF, HF), jnp.float32),
)


def kernel(x, edge_index, W1, b1, W2, b2):
    src = edge_index[0]
    dst = edge_index[1]
    pad_idx = (jnp.arange(E_PAD - E, dtype=jnp.int32) % (N_PAD - N)) + N
    src_p = jnp.concatenate([src, pad_idx]).reshape(NW, CPT, CH)
    dst_p = jnp.concatenate([dst, pad_idx]).reshape(NW, CPT, CH)

    zeros_f = jnp.zeros((STAGE_ROWS, HID), jnp.float32)
    ones_r = jnp.ones((CH, HID), jnp.float32)
    dummy_t = jnp.zeros((N_PAD, HID), jnp.float32)
    mode_mp = jnp.zeros((16,), jnp.float32)
    mode_deg = jnp.ones((16,), jnp.float32)

    degf = _sc_message_pass(dummy_t, src_p, dst_p, zeros_f, ones_r,
                            mode_deg)

    degff = degf.reshape(NC, NF, HF)

    h1 = _tc1(x, degf, W1)
    agg1 = _sc_message_pass(h1, src_p, dst_p, zeros_f, ones_r, mode_mp)
    h2f = _tc2(agg1.reshape(NC, NF, HF), degff, degff,
               jnp.tile(b1, 4).reshape(1, HF), jnp.kron(jnp.eye(4), W2))
    h2 = h2f.reshape(N_PAD, HID)
    agg2 = _sc_message_pass(h2, src_p, dst_p, zeros_f, ones_r, mode_mp)
    outf = _tc3(agg2.reshape(NC, NF, HF), degff,
                jnp.tile(b2, 4).reshape(1, HF))
    return outf.reshape(N_PAD, HID)[:N]


# speedup vs baseline: 1.6822x; 1.0886x over previous
import functools

import jax
import jax.numpy as jnp
from jax import lax
from jax.experimental import pallas as pl
from jax.experimental.pallas import tpu as pltpu
from jax.experimental.pallas import tpu_sc as plsc

N = 10000
E = 320000
IN_FEATS = 128
HID = 32

NC = 2
NS = 16
NW = NC * NS
CH = 128
CPT = 79
EPT = CH * CPT
E_PAD = NW * EPT
N_PAD = 10240
ROWS_PER_TILE = N_PAD // NS
STAGE_ROWS = 160
N_SUB = ROWS_PER_TILE // STAGE_ROWS

_mesh = plsc.VectorSubcoreMesh(core_axis_name="c", subcore_axis_name="s")


@functools.partial(
    pl.kernel,
    mesh=_mesh,
    compiler_params=pltpu.CompilerParams(use_tc_tiling_on_sc=False, needs_layout_passes=False),
    out_type=jax.ShapeDtypeStruct((NC, N_PAD, HID), jnp.float32),
    scratch_types=[
        pltpu.VMEM((CPT, CH), jnp.int32),
        pltpu.VMEM((CPT, CH), jnp.int32),
        pltpu.VMEM((CH, HID), jnp.float32),
        pltpu.VMEM((CH, HID), jnp.float32),
        pltpu.VMEM((STAGE_ROWS, HID), jnp.float32),
        pltpu.VMEM((16,), jnp.float32),
        pltpu.VMEM_SHARED((N_PAD, HID), jnp.float32),
        pltpu.SemaphoreType.DMA,
        pltpu.SemaphoreType.DMA,
    ],
)
def _sc_message_pass(h_hbm, src_hbm, dst_hbm, zeros_hbm, ones_hbm, mode_hbm,
                     out_hbm, srcv, dstv, rows_a, rows_b, stage_v, mode_v,
                     agg_sh, sem_a, sem_b):
    cid = lax.axis_index("c")
    sid = lax.axis_index("s")
    wid = sid * NC + cid
    r0 = sid * ROWS_PER_TILE
    pltpu.sync_copy(zeros_hbm, stage_v)
    for k in range(N_SUB):
        pltpu.sync_copy(stage_v,
                        agg_sh.at[pl.ds(r0 + k * STAGE_ROWS, STAGE_ROWS)])
    pltpu.sync_copy(mode_hbm, mode_v)
    pltpu.sync_copy(dst_hbm.at[wid], dstv)
    is_deg = jnp.sum(mode_v[...]) > 0.5
    plsc.subcore_barrier()

    @pl.when(jnp.logical_not(is_deg))
    def _mp_loop():
        pltpu.sync_copy(src_hbm.at[wid], srcv)
        pltpu.make_async_copy(h_hbm.at[srcv.at[0]], rows_a, sem_a).start()

        def pair(i, c):
            j = 1 + 2 * i
            pltpu.make_async_copy(h_hbm.at[srcv.at[j]], rows_b,
                                  sem_b).start()
            pltpu.make_async_copy(h_hbm.at[srcv.at[j - 1]], rows_a,
                                  sem_a).wait()
            pltpu.sync_copy(rows_a, agg_sh.at[dstv.at[j - 1]], add=True)
            pltpu.make_async_copy(h_hbm.at[srcv.at[j + 1]], rows_a,
                                  sem_a).start()
            pltpu.make_async_copy(h_hbm.at[srcv.at[j]], rows_b,
                                  sem_b).wait()
            pltpu.sync_copy(rows_b, agg_sh.at[dstv.at[j]], add=True)
            return c

        lax.fori_loop(0, (CPT - 1) // 2, pair, 0)
        pltpu.make_async_copy(h_hbm.at[srcv.at[CPT - 1]], rows_a,
                              sem_a).wait()
        pltpu.sync_copy(rows_a, agg_sh.at[dstv.at[CPT - 1]], add=True)

    @pl.when(is_deg)
    def _deg_loop():
        pltpu.sync_copy(ones_hbm, rows_a)

        def scatter_shard(idx_hbm, w):
            pltpu.sync_copy(idx_hbm.at[w], dstv)

            def dpair(i, c):
                j = 2 * i
                sa = pltpu.make_async_copy(rows_a, agg_sh.at[dstv.at[j]],
                                           sem_a)
                sb = pltpu.make_async_copy(rows_a, agg_sh.at[dstv.at[j + 1]],
                                           sem_b)
                sa.start(add=True)
                sb.start(add=True)
                sa.wait()
                sb.wait()
                return c

            lax.fori_loop(0, (CPT - 1) // 2, dpair, 0)
            pltpu.sync_copy(rows_a, agg_sh.at[dstv.at[CPT - 1]], add=True)

        @pl.when(cid == 0)
        def _in_degrees():
            scatter_shard(dst_hbm, sid)
            scatter_shard(dst_hbm, sid + NS)

        @pl.when(cid == 1)
        def _out_degrees():
            scatter_shard(src_hbm, sid)
            scatter_shard(src_hbm, sid + NS)

    plsc.subcore_barrier()
    for k in range(N_SUB):
        sub = pl.ds(r0 + k * STAGE_ROWS, STAGE_ROWS)
        pltpu.sync_copy(agg_sh.at[sub], stage_v)
        pltpu.sync_copy(stage_v, out_hbm.at[cid, sub])


RB = 1000


def _gated_rsqrt(d):
    return jnp.where(d > 0.0, lax.rsqrt(jnp.maximum(d, 1.0)), 0.0)


def _tc1_body(x_ref, w_ref, h_ref):
    h_ref[...] = jnp.dot(x_ref[...], w_ref[...],
                         preferred_element_type=jnp.float32)


def _tcs_body(h_ref, dout_ref, o_ref):
    o_ref[...] = h_ref[...] * _gated_rsqrt(dout_ref[...][0])


def _tc2_body(p_ref, din_ref, dout_ref, b1_ref, w2_ref, h2_ref):
    p = p_ref[...]
    ndst = _gated_rsqrt(din_ref[...][0])
    nsrc = _gated_rsqrt(dout_ref[...][0])
    h = (p[0] + p[1]) * ndst + b1_ref[...]
    h = jnp.maximum(h, 0.0)
    h = h * nsrc
    h2_ref[...] = jnp.dot(h, w2_ref[...], preferred_element_type=jnp.float32)


def _tc3_body(p_ref, din_ref, b2_ref, o_ref):
    p = p_ref[...]
    ndst = _gated_rsqrt(din_ref[...][0])
    o_ref[...] = (p[0] + p[1]) * ndst + b2_ref[...]


_G = N // RB
NF = N_PAD // 4
HF = 4 * HID
RBF = 512
_GF = NF // RBF

_tc1 = pl.pallas_call(
    _tc1_body,
    grid=(_G,),
    in_specs=[
        pl.BlockSpec((RB, IN_FEATS), lambda i: (i, 0)),
        pl.BlockSpec((IN_FEATS, HID), lambda i: (0, 0)),
    ],
    out_specs=pl.BlockSpec((RB, HID), lambda i: (i, 0)),
    out_shape=jax.ShapeDtypeStruct((N_PAD, HID), jnp.float32),
)

_tcs = pl.pallas_call(
    _tcs_body,
    grid=(NF // RBF,),
    in_specs=[
        pl.BlockSpec((RBF, HF), lambda i: (i, 0)),
        pl.BlockSpec((1, RBF, HF), lambda i: (1, i, 0)),
    ],
    out_specs=pl.BlockSpec((RBF, HF), lambda i: (i, 0)),
    out_shape=jax.ShapeDtypeStruct((NF, HF), jnp.float32),
)

_tc2 = pl.pallas_call(
    _tc2_body,
    grid=(_GF,),
    in_specs=[
        pl.BlockSpec((NC, RBF, HF), lambda i: (0, i, 0)),
        pl.BlockSpec((1, RBF, HF), lambda i: (0, i, 0)),
        pl.BlockSpec((1, RBF, HF), lambda i: (1, i, 0)),
        pl.BlockSpec((1, HF), lambda i: (0, 0)),
        pl.BlockSpec((HF, HF), lambda i: (0, 0)),
    ],
    out_specs=pl.BlockSpec((RBF, HF), lambda i: (i, 0)),
    out_shape=jax.ShapeDtypeStruct((NF, HF), jnp.float32),
)

_tc3 = pl.pallas_call(
    _tc3_body,
    grid=(_GF,),
    in_specs=[
        pl.BlockSpec((NC, RBF, HF), lambda i: (0, i, 0)),
        pl.BlockSpec((1, RBF, HF), lambda i: (0, i, 0)),
        pl.BlockSpec((1, HF), lambda i: (0, 0)),
    ],
    out_specs=pl.BlockSpec((RBF, HF), lambda i: (i, 0)),
    out_shape=jax.ShapeDtypeStruct((NF, HF), jnp.float32),
)


def kernel(x, edge_index, W1, b1, W2, b2):
    src = edge_index[0]
    dst = edge_index[1]
    pad_idx = (jnp.arange(E_PAD - E, dtype=jnp.int32) % (N_PAD - N)) + N
    src_p = jnp.concatenate([src, pad_idx]).reshape(NW, CPT, CH)
    dst_p = jnp.concatenate([dst, pad_idx]).reshape(NW, CPT, CH)

    zeros_f = jnp.zeros((STAGE_ROWS, HID), jnp.float32)
    ones_r = jnp.ones((CH, HID), jnp.float32)
    dummy_t = jnp.zeros((N_PAD, HID), jnp.float32)
    mode_mp = jnp.zeros((16,), jnp.float32)
    mode_deg = jnp.ones((16,), jnp.float32)

    degf = _sc_message_pass(dummy_t, src_p, dst_p, zeros_f, ones_r,
                            mode_deg)

    degff = degf.reshape(NC, NF, HF)

    mm1 = _tc1(x, W1)
    h1 = _tcs(mm1.reshape(NF, HF), degff).reshape(N_PAD, HID)
    agg1 = _sc_message_pass(h1, src_p, dst_p, zeros_f, ones_r, mode_mp)
    h2f = _tc2(agg1.reshape(NC, NF, HF), degff, degff,
               jnp.tile(b1, 4).reshape(1, HF), jnp.kron(jnp.eye(4), W2))
    h2 = h2f.reshape(N_PAD, HID)
    agg2 = _sc_message_pass(h2, src_p, dst_p, zeros_f, ones_r, mode_mp)
    outf = _tc3(agg2.reshape(NC, NF, HF), degff,
                jnp.tile(b2, 4).reshape(1, HF))
    return outf.reshape(N_PAD, HID)[:N]
